# baseline (speedup 1.0000x reference)
import functools

import jax
import jax.numpy as jnp
from jax import lax
from jax.experimental import pallas as pl
from jax.experimental.pallas import tpu as pltpu
from jax.experimental.pallas import tpu_sc as plsc

B = 4096 * 26
NC, NS, L = 2, 16, 16
NW = NC * NS
TPW = B // NW
CH = 128
NCHUNK = TPW // CH
NG = CH // L
NPAIR = NCHUNK // 2 - 1

GSTRIDE = 129
GSZ = 2 * CH * GSTRIDE
CSTRIDE = 65


def _tc_fold01(a_ref, b_ref, o_ref):
    o_ref[...] = jnp.dot(a_ref[...], b_ref[...],
                         preferred_element_type=jnp.float32)


def _div100(v):
    return ((v.astype(jnp.float32) + 0.5) * jnp.float32(0.01)).astype(jnp.int32)


_mesh = plsc.VectorSubcoreMesh(core_axis_name="c", subcore_axis_name="s")


@functools.partial(
    pl.kernel,
    mesh=_mesh,
    compiler_params=pltpu.CompilerParams(needs_layout_passes=False),
    out_type=jax.ShapeDtypeStruct((B * 64,), jnp.float32),
    scratch_types=[
        pltpu.VMEM((100 * CSTRIDE,), jnp.float32),
        pltpu.VMEM((CH,), jnp.int32),
        pltpu.VMEM((2, 2, CH), jnp.int32),
        pltpu.VMEM((2, CH), jnp.int32),
        pltpu.VMEM((2, 2, CH, 128), jnp.float32),
        pltpu.VMEM((GSZ,), jnp.float32),
        pltpu.VMEM((CH * 64,), jnp.float32),
        pltpu.SemaphoreType.DMA,
        pltpu.SemaphoreType.DMA,
    ],
)
def _sc_contract(table_hbm, idx_hbm, c2_hbm, out_hbm,
                 c2_v, idx_v, gidx_v, i2_v, g_v, gp_v, out_v, sem0, sem1):
    wid = lax.axis_index("s") * NC + lax.axis_index("c")
    tok0 = wid * TPW
    pltpu.sync_copy(c2_hbm, c2_v)
    iota = lax.iota(jnp.int32, L)
    iota129 = iota * GSTRIDE
    iota64 = iota * 64
    sems = (sem0, sem1)

    def prefetch(ck, buf):
        t0 = tok0 + ck * CH
        pltpu.sync_copy(idx_hbm.at[pl.ds(t0, CH)], idx_v)

        def build(gi, c):
            sl = pl.ds(gi * L, L)
            v = idx_v[sl]
            q = _div100(v)
            i2 = v - q * 100
            i0 = _div100(q)
            i1 = q - i0 * 100
            base = i0 * 200 + i1
            gidx_v[buf, 0, sl] = base
            gidx_v[buf, 1, sl] = base + 100
            i2_v[buf, sl] = i2
            return c

        lax.fori_loop(0, NG, build, 0)
        for p in range(2):
            pltpu.async_copy(table_hbm.at[gidx_v.at[buf, p]],
                             g_v.at[buf, p], sems[buf])

    def drain(buf):
        for p in range(2):
            pltpu.make_async_copy(table_hbm.at[gidx_v.at[buf, p]],
                                  g_v.at[buf, p], sems[buf]).wait()

    def compute(ck, buf):
        t0 = tok0 + ck * CH

        def cprow(r4, c):
            r = r4 * 4
            for dr in range(4):
                rb = iota + (r + dr) * GSTRIDE
                for p in range(2):
                    for j in range(8):
                        plsc.store_scatter(
                            gp_v, [rb + (p * CH * GSTRIDE + j * L)],
                            g_v[buf, p, r + dr, pl.ds(j * L, L)])
            return c

        with jax.named_scope("copyloop"):
            lax.fori_loop(0, CH // 4, cprow, 0)

        def group(gi, c):
            tok129 = iota129 + gi * (L * GSTRIDE)
            tok64 = iota64 + gi * (L * 64)
            i2v65 = i2_v[buf, pl.ds(gi * L, L)] * CSTRIDE
            for o0 in range(4):
                poff = (o0 // 2) * (CH * GSTRIDE)
                cbase = (o0 % 2) * 64
                accs = [None] * 16
                for r2 in range(16):
                    ms = [plsc.load_gather(c2_v, [i2v65 + (r2 * 4 + o2)])
                          for o2 in range(4)]
                    gs = [plsc.load_gather(
                              gp_v,
                              [tok129 + (poff + cbase + o1 * 16 + r2)])
                          for o1 in range(4)]
                    for o1 in range(4):
                        for o2 in range(4):
                            prod = gs[o1] * ms[o2]
                            k = o1 * 4 + o2
                            accs[k] = prod if r2 == 0 else accs[k] + prod
                for o1 in range(4):
                    for o2 in range(4):
                        plsc.store_scatter(
                            out_v, [tok64 + (o0 * 16 + o1 * 4 + o2)],
                            accs[o1 * 4 + o2])
            return c

        with jax.named_scope("grouploop"):
            lax.fori_loop(0, NG, group, 0)
        with jax.named_scope("outcopy"):
            pltpu.sync_copy(out_v, out_hbm.at[pl.ds(t0 * 64, CH * 64)])

    prefetch(0, 0)

    def pair_body(k, carry):
        ck = 2 * k
        prefetch(ck + 1, 1)
        drain(0)
        compute(ck, 0)
        prefetch(ck + 2, 0)
        drain(1)
        compute(ck + 1, 1)
        return carry

    lax.fori_loop(0, NPAIR, pair_body, 0)
    prefetch(NCHUNK - 1, 1)
    drain(0)
    compute(NCHUNK - 2, 0)
    drain(1)
    compute(NCHUNK - 1, 1)


def kernel(indices, core0, core1, core2):
    w01 = pl.pallas_call(
        _tc_fold01,
        out_shape=jax.ShapeDtypeStruct((400, 6400), jnp.float32),
    )(core0.reshape(400, 16), core1.reshape(16, 6400))
    table = (w01.reshape(100, 2, 2, 100, 64)
                .transpose(0, 1, 3, 2, 4)
                .reshape(20000, 128))
    c2p = jnp.transpose(core2, (1, 0, 2, 3)).reshape(100, 64)
    c2p = jnp.pad(c2p, ((0, 0), (0, CSTRIDE - 64))).reshape(-1)
    idx = indices.reshape(-1)
    out = _sc_contract(table, idx, c2p)
    return out.reshape(B, 64)


# speedup vs baseline: 1.0251x; 1.0251x over previous
import functools

import jax
import jax.numpy as jnp
from jax import lax
from jax.experimental import pallas as pl
from jax.experimental.pallas import tpu as pltpu
from jax.experimental.pallas import tpu_sc as plsc

B = 4096 * 26
NC, NS, L = 2, 16, 16
NW = NC * NS
TPW = B // NW
CH = 128
NCHUNK = TPW // CH
NG = CH // L
NPAIR = NCHUNK // 2 - 1

GSTRIDE = 257
GSZ = CH * GSTRIDE
CSTRIDE = 65


def _tc_fold01(a_ref, b_ref, o_ref):
    c = jnp.dot(a_ref[0], b_ref[...], preferred_element_type=jnp.float32)
    t = c.reshape(2, 2, 100, 64).transpose(2, 0, 1, 3)
    o_ref[0] = t.reshape(100, 256)


def _div100(v):
    return ((v.astype(jnp.float32) + 0.5) * jnp.float32(0.01)).astype(jnp.int32)


_mesh = plsc.VectorSubcoreMesh(core_axis_name="c", subcore_axis_name="s")


@functools.partial(
    pl.kernel,
    mesh=_mesh,
    compiler_params=pltpu.CompilerParams(needs_layout_passes=False),
    out_type=jax.ShapeDtypeStruct((B * 64,), jnp.float32),
    scratch_types=[
        pltpu.VMEM((6400,), jnp.float32),
        pltpu.VMEM((100 * CSTRIDE,), jnp.float32),
        pltpu.VMEM((CH,), jnp.int32),
        pltpu.VMEM((2, CH), jnp.int32),
        pltpu.VMEM((2, CH), jnp.int32),
        pltpu.VMEM((2, CH, 256), jnp.float32),
        pltpu.VMEM((GSZ,), jnp.float32),
        pltpu.VMEM((CH * 64,), jnp.float32),
        pltpu.SemaphoreType.DMA,
        pltpu.SemaphoreType.DMA,
    ],
)
def _sc_contract(table_hbm, idx_hbm, c2_hbm, out_hbm,
                 c2r_v, c2_v, idx_v, gidx_v, i2_v, g_v, gp_v, out_v,
                 sem0, sem1):
    wid = lax.axis_index("s") * NC + lax.axis_index("c")
    tok0 = wid * TPW
    pltpu.sync_copy(c2_hbm, c2r_v)
    iota = lax.iota(jnp.int32, L)
    iota64 = iota * 64
    sems = (sem0, sem1)

    def c2stage(r, c):
        rb = iota + r * CSTRIDE
        for j in range(4):
            plsc.store_scatter(c2_v, [rb + j * L], c2r_v[pl.ds(r * 64 + j * L, L)])
        return c

    lax.fori_loop(0, 100, c2stage, 0)

    def prefetch(ck, buf):
        t0 = tok0 + ck * CH
        pltpu.sync_copy(idx_hbm.at[pl.ds(t0, CH)], idx_v)

        def build(gi, c):
            sl = pl.ds(gi * L, L)
            v = idx_v[sl]
            q = _div100(v)
            gidx_v[buf, sl] = q
            i2_v[buf, sl] = v - q * 100
            return c

        lax.fori_loop(0, NG, build, 0)
        pltpu.async_copy(table_hbm.at[gidx_v.at[buf]], g_v.at[buf], sems[buf])

    def drain(buf):
        pltpu.make_async_copy(table_hbm.at[gidx_v.at[buf]],
                              g_v.at[buf], sems[buf]).wait()

    def compute(ck, buf):
        t0 = tok0 + ck * CH

        def cprow(r4, c):
            r = r4 * 4
            for dr in range(4):
                rb = iota + (r + dr) * GSTRIDE
                for j in range(16):
                    plsc.store_scatter(
                        gp_v, [rb + j * L],
                        g_v[buf, r + dr, pl.ds(j * L, L)])
            return c

        lax.fori_loop(0, CH // 4, cprow, 0)

        def group(gi, c):
            tok257 = (iota + gi * L) * GSTRIDE
            tok64 = iota64 + gi * (L * 64)
            i2v65 = i2_v[buf, pl.ds(gi * L, L)] * CSTRIDE
            for o0 in range(4):
                goff = (o0 // 2) * 128 + (o0 % 2) * 64
                accs = [None] * 16
                for r2 in range(16):
                    ms = [plsc.load_gather(c2_v, [i2v65 + (r2 * 4 + o2)])
                          for o2 in range(4)]
                    gs = [plsc.load_gather(
                              gp_v, [tok257 + (goff + o1 * 16 + r2)])
                          for o1 in range(4)]
                    for o1 in range(4):
                        for o2 in range(4):
                            prod = gs[o1] * ms[o2]
                            k = o1 * 4 + o2
                            accs[k] = prod if r2 == 0 else accs[k] + prod
                for o1 in range(4):
                    for o2 in range(4):
                        plsc.store_scatter(
                            out_v, [tok64 + (o0 * 16 + o1 * 4 + o2)],
                            accs[o1 * 4 + o2])
            return c

        lax.fori_loop(0, NG, group, 0)
        pltpu.sync_copy(out_v, out_hbm.at[pl.ds(t0 * 64, CH * 64)])

    prefetch(0, 0)

    def pair_body(k, carry):
        ck = 2 * k
        prefetch(ck + 1, 1)
        drain(0)
        compute(ck, 0)
        prefetch(ck + 2, 0)
        drain(1)
        compute(ck + 1, 1)
        return carry

    lax.fori_loop(0, NPAIR, pair_body, 0)
    prefetch(NCHUNK - 1, 1)
    drain(0)
    compute(NCHUNK - 2, 0)
    drain(1)
    compute(NCHUNK - 1, 1)


def kernel(indices, core0, core1, core2):
    table = pl.pallas_call(
        _tc_fold01,
        grid=(100,),
        in_specs=[pl.BlockSpec((1, 4, 16), lambda i: (i, 0, 0)),
                  pl.BlockSpec((16, 6400), lambda i: (0, 0))],
        out_specs=pl.BlockSpec((1, 100, 256), lambda i: (i, 0, 0)),
        out_shape=jax.ShapeDtypeStruct((100, 100, 256), jnp.float32),
    )(core0.reshape(100, 4, 16), core1.reshape(16, 6400))
    table = table.reshape(10000, 256)
    c2p = jnp.transpose(core2, (1, 0, 2, 3)).reshape(-1)
    idx = indices.reshape(-1)
    out = _sc_contract(table, idx, c2p)
    return out.reshape(B, 64)


# speedup vs baseline: 1.1242x; 1.0967x over previous
import functools

import jax
import jax.numpy as jnp
from jax import lax
from jax.experimental import pallas as pl
from jax.experimental.pallas import tpu as pltpu
from jax.experimental.pallas import tpu_sc as plsc

B = 4096 * 26
NC, NS, L = 2, 16, 16
NW = NC * NS
TPW = B // NW
CH = 128
NCHUNK = TPW // CH
NG = CH // L
NPAIR = NCHUNK // 2 - 1

GSTRIDE = 257
GSZ = CH * GSTRIDE
CSTRIDE = 65


def _tc_fold01(a_ref, b_ref, o_ref):
    c = jnp.dot(a_ref[...], b_ref[...], preferred_element_type=jnp.float32)
    t = c.reshape(2, 2, 2, 100, 64).transpose(0, 3, 1, 2, 4)
    o_ref[...] = t.reshape(200, 256)


def _div100(v):
    return ((v.astype(jnp.float32) + 0.5) * jnp.float32(0.01)).astype(jnp.int32)


_mesh = plsc.VectorSubcoreMesh(core_axis_name="c", subcore_axis_name="s")


@functools.partial(
    pl.kernel,
    mesh=_mesh,
    compiler_params=pltpu.CompilerParams(needs_layout_passes=False),
    out_type=jax.ShapeDtypeStruct((B * 64,), jnp.float32),
    scratch_types=[
        pltpu.VMEM((100 * CSTRIDE,), jnp.float32),
        pltpu.VMEM((2 * CH,), jnp.int32),
        pltpu.VMEM((2, CH), jnp.int32),
        pltpu.VMEM((2, CH), jnp.int32),
        pltpu.VMEM((2, CH, 256), jnp.float32),
        pltpu.VMEM((GSZ,), jnp.float32),
        pltpu.VMEM((2 * CH * 64,), jnp.float32),
        pltpu.SemaphoreType.DMA,
        pltpu.SemaphoreType.DMA,
        pltpu.SemaphoreType.DMA,
        pltpu.SemaphoreType.DMA,
        pltpu.SemaphoreType.DMA,
        pltpu.SemaphoreType.DMA,
    ],
)
def _sc_contract(table_hbm, idx_hbm, c2_hbm, out_hbm,
                 c2_v, idx_v, gidx_v, i2_v, g_v, gp_v, out_v,
                 semg0, semg1, semi0, semi1, semo0, semo1):
    wid = lax.axis_index("s") * NC + lax.axis_index("c")
    tok0 = wid * TPW
    pltpu.sync_copy(c2_hbm, gp_v.at[pl.ds(0, 6400)])
    iota = lax.iota(jnp.int32, L)
    iota64 = iota * 64
    semg = (semg0, semg1)
    semi = (semi0, semi1)
    semo = (semo0, semo1)

    def c2stage(r, c):
        rb = iota + r * CSTRIDE
        for j in range(4):
            plsc.store_scatter(c2_v, [rb + j * L],
                               gp_v[pl.ds(r * 64 + j * L, L)])
        return c

    lax.fori_loop(0, 100, c2stage, 0)

    def fire_idx(ck, buf):
        t0 = tok0 + ck * CH
        pltpu.async_copy(idx_hbm.at[pl.ds(t0, CH)],
                         idx_v.at[pl.ds(buf * CH, CH)], semi[buf])

    def drain_idx(ck, buf):
        t0 = tok0 + ck * CH
        pltpu.make_async_copy(idx_hbm.at[pl.ds(t0, CH)],
                              idx_v.at[pl.ds(buf * CH, CH)], semi[buf]).wait()

    def prefetch(ck, buf):
        drain_idx(ck, buf)

        def build(gi, c):
            sl = pl.ds(gi * L, L)
            v = idx_v[pl.ds(buf * CH + gi * L, L)]
            q = _div100(v)
            gidx_v[buf, sl] = q
            i2_v[buf, sl] = v - q * 100
            return c

        lax.fori_loop(0, NG, build, 0)
        pltpu.async_copy(table_hbm.at[gidx_v.at[buf]], g_v.at[buf], semg[buf])
        fire_idx((ck + 2) % NCHUNK, buf)

    def drain_gather(buf):
        pltpu.make_async_copy(table_hbm.at[gidx_v.at[buf]],
                              g_v.at[buf], semg[buf]).wait()

    def compute(ck, buf, drain_out):
        t0 = tok0 + ck * CH

        def cprow(r8, c):
            r = r8 * 8
            for dr in range(8):
                rb = iota + (r + dr) * GSTRIDE
                for j in range(16):
                    plsc.store_scatter(
                        gp_v, [rb + j * L],
                        g_v[buf, r + dr, pl.ds(j * L, L)])
            return c

        lax.fori_loop(0, CH // 8, cprow, 0)

        def _drain_prev_out():
            pltpu.make_async_copy(
                out_v.at[pl.ds(buf * CH * 64, CH * 64)],
                out_hbm.at[pl.ds((tok0 + (ck - 2) * CH) * 64, CH * 64)],
                semo[buf]).wait()

        if isinstance(drain_out, bool):
            if drain_out:
                _drain_prev_out()
        else:
            pl.when(drain_out)(_drain_prev_out)

        def group(gi, c):
            tok257 = (iota + gi * L) * GSTRIDE
            tok64 = iota64 + gi * (L * 64)
            i2v65 = i2_v[buf, pl.ds(gi * L, L)] * CSTRIDE
            for o0 in range(4):
                goff = o0 * 64
                accs = [None] * 16
                for r2 in range(16):
                    ms = [plsc.load_gather(c2_v, [i2v65 + (r2 * 4 + o2)])
                          for o2 in range(4)]
                    gs = [plsc.load_gather(
                              gp_v, [tok257 + (goff + o1 * 16 + r2)])
                          for o1 in range(4)]
                    for o1 in range(4):
                        for o2 in range(4):
                            prod = gs[o1] * ms[o2]
                            k = o1 * 4 + o2
                            accs[k] = prod if r2 == 0 else accs[k] + prod
                for o1 in range(4):
                    for o2 in range(4):
                        plsc.store_scatter(
                            out_v,
                            [tok64 + (buf * CH * 64 + o0 * 16 + o1 * 4 + o2)],
                            accs[o1 * 4 + o2])
            return c

        lax.fori_loop(0, NG, group, 0)
        pltpu.async_copy(out_v.at[pl.ds(buf * CH * 64, CH * 64)],
                         out_hbm.at[pl.ds(t0 * 64, CH * 64)], semo[buf])

    fire_idx(0, 0)
    fire_idx(1, 1)
    prefetch(0, 0)

    def pair_body(k, carry):
        ck = 2 * k
        prefetch(ck + 1, 1)
        drain_gather(0)
        compute(ck, 0, k > 0)

        @pl.when(k < NCHUNK // 2 - 1)
        def _():
            prefetch(ck + 2, 0)

        drain_gather(1)
        compute(ck + 1, 1, k > 0)
        return carry

    lax.fori_loop(0, NCHUNK // 2, pair_body, 0)
    for buf, ck in ((0, NCHUNK - 2), (1, NCHUNK - 1)):
        pltpu.make_async_copy(
            out_v.at[pl.ds(buf * CH * 64, CH * 64)],
            out_hbm.at[pl.ds((tok0 + ck * CH) * 64, CH * 64)],
            semo[buf]).wait()
    drain_idx(0, 0)
    drain_idx(1, 1)


def kernel(indices, core0, core1, core2):
    table = pl.pallas_call(
        _tc_fold01,
        grid=(50,),
        in_specs=[pl.BlockSpec((8, 16), lambda i: (i, 0)),
                  pl.BlockSpec((16, 6400), lambda i: (0, 0))],
        out_specs=pl.BlockSpec((200, 256), lambda i: (i, 0)),
        out_shape=jax.ShapeDtypeStruct((10000, 256), jnp.float32),
    )(core0.reshape(400, 16), core1.reshape(16, 6400))
    c2p = jnp.transpose(core2, (1, 0, 2, 3)).reshape(-1)
    idx = indices.reshape(-1)
    out = _sc_contract(table, idx, c2p)
    return out.reshape(B, 64)


# speedup vs baseline: 1.3287x; 1.1819x over previous
import functools

import jax
import jax.numpy as jnp
from jax import lax
from jax.experimental import pallas as pl
from jax.experimental.pallas import tpu as pltpu
from jax.experimental.pallas import tpu_sc as plsc

B = 4096 * 26
NC, NS, L = 2, 16, 16
NW = NC * NS
TPW = B // NW
CH = 128
NCHUNK = TPW // CH
NG = CH // L
NPAIR = NCHUNK // 2 - 1

GSTRIDE = 257
GSZ = CH * GSTRIDE
CSTRIDE = 65


def _tc_fold01(a_ref, b_ref, o_ref):
    c = jnp.dot(a_ref[...], b_ref[...], preferred_element_type=jnp.float32)
    t = c.reshape(2, 2, 2, 100, 64).transpose(0, 3, 1, 2, 4)
    o_ref[...] = t.reshape(200, 256)


def _div100(v):
    return ((v.astype(jnp.float32) + 0.5) * jnp.float32(0.01)).astype(jnp.int32)


_mesh = plsc.VectorSubcoreMesh(core_axis_name="c", subcore_axis_name="s")


@functools.partial(
    pl.kernel,
    mesh=_mesh,
    compiler_params=pltpu.CompilerParams(needs_layout_passes=False),
    out_type=jax.ShapeDtypeStruct((B * 64,), jnp.float32),
    scratch_types=[
        pltpu.VMEM((100 * CSTRIDE,), jnp.float32),
        pltpu.VMEM((2 * CH,), jnp.int32),
        pltpu.VMEM((2, CH), jnp.int32),
        pltpu.VMEM((2, CH), jnp.int32),
        pltpu.VMEM((2, CH, 256), jnp.float32),
        pltpu.VMEM((GSZ,), jnp.float32),
        pltpu.VMEM((2 * CH * 64,), jnp.float32),
        pltpu.SemaphoreType.DMA,
        pltpu.SemaphoreType.DMA,
        pltpu.SemaphoreType.DMA,
        pltpu.SemaphoreType.DMA,
        pltpu.SemaphoreType.DMA,
        pltpu.SemaphoreType.DMA,
    ],
)
def _sc_contract(table_hbm, idx_hbm, c2_hbm, out_hbm,
                 c2_v, idx_v, gidx_v, i2_v, g_v, gp_v, out_v,
                 semg0, semg1, semi0, semi1, semo0, semo1):
    wid = lax.axis_index("s") * NC + lax.axis_index("c")
    tok0 = wid * TPW
    pltpu.sync_copy(c2_hbm, gp_v.at[pl.ds(0, 6400)])
    iota = lax.iota(jnp.int32, L)
    iota64 = iota * 64
    semg = (semg0, semg1)
    semi = (semi0, semi1)
    semo = (semo0, semo1)

    def c2stage(r, c):
        rb = iota + r * CSTRIDE
        for j in range(4):
            plsc.store_scatter(c2_v, [rb + j * L],
                               gp_v[pl.ds(r * 64 + j * L, L)])
        return c

    lax.fori_loop(0, 100, c2stage, 0)

    def fire_idx(ck, buf):
        t0 = tok0 + ck * CH
        pltpu.async_copy(idx_hbm.at[pl.ds(t0, CH)],
                         idx_v.at[pl.ds(buf * CH, CH)], semi[buf])

    def drain_idx(ck, buf):
        t0 = tok0 + ck * CH
        pltpu.make_async_copy(idx_hbm.at[pl.ds(t0, CH)],
                              idx_v.at[pl.ds(buf * CH, CH)], semi[buf]).wait()

    def prefetch(ck, buf):
        drain_idx(ck, buf)

        @plsc.parallel_loop(0, NG, 1, unroll=2)
        def build(gi):
            sl = pl.ds(gi * L, L)
            v = idx_v[pl.ds(buf * CH + gi * L, L)]
            q = _div100(v)
            gidx_v[buf, sl] = q
            i2_v[buf, sl] = v - q * 100
        pltpu.async_copy(table_hbm.at[gidx_v.at[buf]], g_v.at[buf], semg[buf])
        fire_idx((ck + 2) % NCHUNK, buf)

    def drain_gather(buf):
        pltpu.make_async_copy(table_hbm.at[gidx_v.at[buf]],
                              g_v.at[buf], semg[buf]).wait()

    def compute(ck, buf, drain_out):
        t0 = tok0 + ck * CH

        @plsc.parallel_loop(0, CH // 4, 1, unroll=2)
        def cprow(r4):
            r = r4 * 4
            for dr in range(4):
                rb = iota + (r + dr) * GSTRIDE
                for j in range(16):
                    plsc.store_scatter(
                        gp_v, [rb + j * L],
                        g_v[buf, r + dr, pl.ds(j * L, L)])

        def _drain_prev_out():
            pltpu.make_async_copy(
                out_v.at[pl.ds(buf * CH * 64, CH * 64)],
                out_hbm.at[pl.ds((tok0 + (ck - 2) * CH) * 64, CH * 64)],
                semo[buf]).wait()

        if isinstance(drain_out, bool):
            if drain_out:
                _drain_prev_out()
        else:
            pl.when(drain_out)(_drain_prev_out)

        @plsc.parallel_loop(0, NG, 1)
        def group(gi):
            tok257 = (iota + gi * L) * GSTRIDE
            tok64 = iota64 + gi * (L * 64)
            i2v65 = i2_v[buf, pl.ds(gi * L, L)] * CSTRIDE
            for o0 in range(4):
                goff = o0 * 64
                accs = [None] * 16
                for r2 in range(16):
                    ms = [plsc.load_gather(c2_v, [i2v65 + (r2 * 4 + o2)])
                          for o2 in range(4)]
                    gs = [plsc.load_gather(
                              gp_v, [tok257 + (goff + o1 * 16 + r2)])
                          for o1 in range(4)]
                    for o1 in range(4):
                        for o2 in range(4):
                            prod = gs[o1] * ms[o2]
                            k = o1 * 4 + o2
                            accs[k] = prod if r2 == 0 else accs[k] + prod
                for o1 in range(4):
                    for o2 in range(4):
                        plsc.store_scatter(
                            out_v,
                            [tok64 + (buf * CH * 64 + o0 * 16 + o1 * 4 + o2)],
                            accs[o1 * 4 + o2])

        pltpu.async_copy(out_v.at[pl.ds(buf * CH * 64, CH * 64)],
                         out_hbm.at[pl.ds(t0 * 64, CH * 64)], semo[buf])

    fire_idx(0, 0)
    fire_idx(1, 1)
    prefetch(0, 0)

    def pair_body(k, carry):
        ck = 2 * k
        prefetch(ck + 1, 1)
        drain_gather(0)
        compute(ck, 0, k > 0)

        @pl.when(k < NCHUNK // 2 - 1)
        def _():
            prefetch(ck + 2, 0)

        drain_gather(1)
        compute(ck + 1, 1, k > 0)
        return carry

    lax.fori_loop(0, NCHUNK // 2, pair_body, 0)
    for buf, ck in ((0, NCHUNK - 2), (1, NCHUNK - 1)):
        pltpu.make_async_copy(
            out_v.at[pl.ds(buf * CH * 64, CH * 64)],
            out_hbm.at[pl.ds((tok0 + ck * CH) * 64, CH * 64)],
            semo[buf]).wait()
    drain_idx(0, 0)
    drain_idx(1, 1)


def kernel(indices, core0, core1, core2):
    table = pl.pallas_call(
        _tc_fold01,
        grid=(50,),
        in_specs=[pl.BlockSpec((8, 16), lambda i: (i, 0)),
                  pl.BlockSpec((16, 6400), lambda i: (0, 0))],
        out_specs=pl.BlockSpec((200, 256), lambda i: (i, 0)),
        out_shape=jax.ShapeDtypeStruct((10000, 256), jnp.float32),
    )(core0.reshape(400, 16), core1.reshape(16, 6400))
    c2p = jnp.transpose(core2, (1, 0, 2, 3)).reshape(-1)
    idx = indices.reshape(-1)
    out = _sc_contract(table, idx, c2p)
    return out.reshape(B, 64)


# speedup vs baseline: 1.3980x; 1.0522x over previous
import functools

import jax
import jax.numpy as jnp
from jax import lax
from jax.experimental import pallas as pl
from jax.experimental.pallas import tpu as pltpu
from jax.experimental.pallas import tpu_sc as plsc

B = 4096 * 26
NC, NS, L = 2, 16, 16
NW = NC * NS
TPW = B // NW
CH = 128
NCHUNK = TPW // CH
NG = CH // L
NPAIR = NCHUNK // 2 - 1

GSTRIDE = 257
GSZ = CH * GSTRIDE
CSTRIDE = 65


def _tc_fold01(a_ref, b_ref, o_ref):
    c = jnp.dot(a_ref[...], b_ref[...], preferred_element_type=jnp.float32)
    t = c.reshape(2, 2, 2, 100, 64).transpose(0, 3, 1, 2, 4)
    o_ref[...] = t.reshape(200, 256)


def _div100(v):
    return ((v.astype(jnp.float32) + 0.5) * jnp.float32(0.01)).astype(jnp.int32)


_mesh = plsc.VectorSubcoreMesh(core_axis_name="c", subcore_axis_name="s")


@functools.partial(
    pl.kernel,
    mesh=_mesh,
    compiler_params=pltpu.CompilerParams(needs_layout_passes=False),
    out_type=jax.ShapeDtypeStruct((B * 64,), jnp.float32),
    scratch_types=[
        pltpu.VMEM((100 * CSTRIDE,), jnp.float32),
        pltpu.VMEM((2 * CH,), jnp.int32),
        pltpu.VMEM((2, CH), jnp.int32),
        pltpu.VMEM((2, CH), jnp.int32),
        pltpu.VMEM((2, CH, 256), jnp.float32),
        pltpu.VMEM((GSZ,), jnp.float32),
        pltpu.VMEM((2 * CH * 64,), jnp.float32),
        pltpu.SemaphoreType.DMA,
        pltpu.SemaphoreType.DMA,
        pltpu.SemaphoreType.DMA,
        pltpu.SemaphoreType.DMA,
        pltpu.SemaphoreType.DMA,
        pltpu.SemaphoreType.DMA,
    ],
)
def _sc_contract(table_hbm, idx_hbm, c2_hbm, out_hbm,
                 c2_v, idx_v, gidx_v, i2_v, g_v, gp_v, out_v,
                 semg0, semg1, semi0, semi1, semo0, semo1):
    wid = lax.axis_index("s") * NC + lax.axis_index("c")
    tok0 = wid * TPW
    pltpu.sync_copy(c2_hbm, gp_v.at[pl.ds(0, 6400)])
    iota = lax.iota(jnp.int32, L)
    iota64 = iota * 64
    semg = (semg0, semg1)
    semi = (semi0, semi1)
    semo = (semo0, semo1)

    def c2stage(r, c):
        rb = iota + r * CSTRIDE
        for j in range(4):
            plsc.store_scatter(c2_v, [rb + j * L],
                               gp_v[pl.ds(r * 64 + j * L, L)])
        return c

    lax.fori_loop(0, 100, c2stage, 0)

    def fire_idx(ck, buf):
        t0 = tok0 + ck * CH
        pltpu.async_copy(idx_hbm.at[pl.ds(t0, CH)],
                         idx_v.at[pl.ds(buf * CH, CH)], semi[buf])

    def drain_idx(ck, buf):
        t0 = tok0 + ck * CH
        pltpu.make_async_copy(idx_hbm.at[pl.ds(t0, CH)],
                              idx_v.at[pl.ds(buf * CH, CH)], semi[buf]).wait()

    def prefetch(ck, buf):
        drain_idx(ck, buf)

        @plsc.parallel_loop(0, NG, 1, unroll=2)
        def build(gi):
            sl = pl.ds(gi * L, L)
            v = idx_v[pl.ds(buf * CH + gi * L, L)]
            q = _div100(v)
            gidx_v[buf, sl] = q
            i2_v[buf, sl] = v - q * 100
        pltpu.async_copy(table_hbm.at[gidx_v.at[buf]], g_v.at[buf], semg[buf])
        fire_idx((ck + 2) % NCHUNK, buf)

    def drain_gather(buf):
        pltpu.make_async_copy(table_hbm.at[gidx_v.at[buf]],
                              g_v.at[buf], semg[buf]).wait()

    def compute(ck, buf, drain_out):
        t0 = tok0 + ck * CH

        @plsc.parallel_loop(0, CH // 4, 1, unroll=2)
        def cprow(r4):
            r = r4 * 4
            for dr in range(4):
                rb = iota + (r + dr) * GSTRIDE
                for j in range(16):
                    plsc.store_scatter(
                        gp_v, [rb + j * L],
                        g_v[buf, r + dr, pl.ds(j * L, L)])

        def _drain_prev_out():
            pltpu.make_async_copy(
                out_v.at[pl.ds(buf * CH * 64, CH * 64)],
                out_hbm.at[pl.ds((tok0 + (ck - 2) * CH) * 64, CH * 64)],
                semo[buf]).wait()

        if isinstance(drain_out, bool):
            if drain_out:
                _drain_prev_out()
        else:
            pl.when(drain_out)(_drain_prev_out)

        @plsc.parallel_loop(0, NG, 1)
        def group(gi):
            tok257 = (iota + gi * L) * GSTRIDE
            tok64 = iota64 + gi * (L * 64)
            i2v65 = i2_v[buf, pl.ds(gi * L, L)] * CSTRIDE
            for pp in range(2):
                accs = [None] * 32
                for r2 in range(16):
                    ms = [plsc.load_gather(c2_v, [i2v65 + (r2 * 4 + o2)])
                          for o2 in range(4)]
                    gs = [plsc.load_gather(
                              gp_v,
                              [tok257 + (pp * 128 + o0h * 64 + o1 * 16 + r2)])
                          for o0h in range(2) for o1 in range(4)]
                    for o0h in range(2):
                        for o1 in range(4):
                            for o2 in range(4):
                                prod = gs[o0h * 4 + o1] * ms[o2]
                                k = o0h * 16 + o1 * 4 + o2
                                accs[k] = prod if r2 == 0 else accs[k] + prod
                for o0h in range(2):
                    for o1 in range(4):
                        for o2 in range(4):
                            plsc.store_scatter(
                                out_v,
                                [tok64 + (buf * CH * 64 + (pp * 2 + o0h) * 16
                                          + o1 * 4 + o2)],
                                accs[o0h * 16 + o1 * 4 + o2])

        pltpu.async_copy(out_v.at[pl.ds(buf * CH * 64, CH * 64)],
                         out_hbm.at[pl.ds(t0 * 64, CH * 64)], semo[buf])

    fire_idx(0, 0)
    fire_idx(1, 1)
    prefetch(0, 0)

    def pair_body(k, carry):
        ck = 2 * k
        prefetch(ck + 1, 1)
        drain_gather(0)
        compute(ck, 0, k > 0)

        @pl.when(k < NCHUNK // 2 - 1)
        def _():
            prefetch(ck + 2, 0)

        drain_gather(1)
        compute(ck + 1, 1, k > 0)
        return carry

    lax.fori_loop(0, NCHUNK // 2, pair_body, 0)
    for buf, ck in ((0, NCHUNK - 2), (1, NCHUNK - 1)):
        pltpu.make_async_copy(
            out_v.at[pl.ds(buf * CH * 64, CH * 64)],
            out_hbm.at[pl.ds((tok0 + ck * CH) * 64, CH * 64)],
            semo[buf]).wait()
    drain_idx(0, 0)
    drain_idx(1, 1)


def kernel(indices, core0, core1, core2):
    table = pl.pallas_call(
        _tc_fold01,
        grid=(50,),
        in_specs=[pl.BlockSpec((8, 16), lambda i: (i, 0)),
                  pl.BlockSpec((16, 6400), lambda i: (0, 0))],
        out_specs=pl.BlockSpec((200, 256), lambda i: (i, 0)),
        out_shape=jax.ShapeDtypeStruct((10000, 256), jnp.float32),
    )(core0.reshape(400, 16), core1.reshape(16, 6400))
    c2p = jnp.transpose(core2, (1, 0, 2, 3)).reshape(-1)
    idx = indices.reshape(-1)
    out = _sc_contract(table, idx, c2p)
    return out.reshape(B, 64)
